# Initial kernel scaffold; baseline (speedup 1.0000x reference)
import jax
import jax.numpy as jnp
from jax.experimental import pallas as pl


def kernel(x, tables):
    raise NotImplementedError("write your pallas kernel here")


# speedup vs baseline: 1.0141x; 1.0141x over previous
import functools

import jax
import jax.numpy as jnp
from jax import lax
from jax.experimental import pallas as pl
from jax.experimental.pallas import tpu as pltpu
from jax.experimental.pallas import tpu_sc as plsc

NUM_FIELDS = 26
VOCAB = 100000
EMB_DIM = 32
BATCH = 4096

NC, NS, L = 2, 16, 16
NW = NC * NS
ROWS = BATCH * NUM_FIELDS
RPW = ROWS // NW
CHUNK = 128
NCHUNK = RPW // CHUNK
HALF = NCHUNK // 2

_mesh = plsc.VectorSubcoreMesh(core_axis_name="c", subcore_axis_name="s")


@functools.partial(
    pl.kernel,
    mesh=_mesh,
    out_type=jax.ShapeDtypeStruct((ROWS, EMB_DIM), jnp.float32),
    scratch_types=[
        pltpu.VMEM((RPW,), jnp.int32),
        pltpu.VMEM((RPW, EMB_DIM), jnp.float32),
        pltpu.SemaphoreType.DMA,
    ],
    compiler_params=pltpu.CompilerParams(use_tc_tiling_on_sc=False),
)
def _sc_gather(x_hbm, table_hbm, out_hbm, idx_v, rows_v, sem):
    wid = lax.axis_index("s") * NC + lax.axis_index("c")
    base = wid * RPW

    pltpu.sync_copy(x_hbm.at[pl.ds(base, RPW)], idx_v)

    lane = lax.iota(jnp.int32, L)

    def add_offsets(t, _):
        pos = t * L + lane
        fld = lax.rem(pos, NUM_FIELDS)
        idx_v[pl.ds(t * L, L)] = idx_v[pl.ds(t * L, L)] + fld * VOCAB
        return _

    lax.fori_loop(0, RPW // L, add_offsets, 0)

    for h in range(2):
        copies = []
        for c in range(HALF):
            off = (h * HALF + c) * CHUNK
            copies.append(
                pltpu.async_copy(
                    table_hbm.at[idx_v.at[pl.ds(off, CHUNK)]],
                    rows_v.at[pl.ds(off, CHUNK)],
                    sem,
                )
            )
        for cp in copies:
            cp.wait()

    pltpu.sync_copy(rows_v, out_hbm.at[pl.ds(base, RPW)])


def kernel(x, tables):
    f, v, d = tables.shape
    x_flat = x.reshape(-1)
    table_flat = tables.reshape(f * v, d)
    out = _sc_gather(x_flat, table_flat)
    return out.reshape(x.shape[0], f, d)
